# baseline (speedup 1.0000x reference)
import jax
import jax.numpy as jnp
from jax import lax
from jax.experimental import pallas as pl
from jax.experimental.pallas import tpu as pltpu
from jax.experimental.pallas import tpu_sc as plsc

B = 16384
N_ROWS = 100000
NB_PAD = 100352
EMB_DIM = 32
BERT_DIM = 768
NC = 2
NS = 16
L = 16
NW = NC * NS
BPW = B // NW
C = 32
NCH = BPW // C
CB = 128
NCHB = BPW // CB
QR = BPW // 128


def _diags():
    iot = lax.iota(jnp.int32, L)
    return [jnp.where(iot + k >= L, iot + k - L, iot + k) for k in range(L)]


def _stage_ids(uid2_h, iid2_h, uidq_v, iidq_v, uid_v, iid_v, wid):
    pltpu.sync_copy(uid2_h.at[pl.ds(wid * QR, QR)], uidq_v)
    pltpu.sync_copy(iid2_h.at[pl.ds(wid * QR, QR)], iidq_v)
    for r in range(QR):
        for k in range(128 // L):
            o = r * 128 + k * L
            uid_v[pl.ds(o, L)] = uidq_v.at[r][pl.ds(k * L, L)]
            iid_v[pl.ds(o, L)] = iidq_v.at[r][pl.ds(k * L, L)]


def _text_body(uid2_h, iid2_h, ut_h, it_h, ubp_h, ibp_h, b16_h, out_h,
               uidq_v, iidq_v, uid_v, iid_v, bu_v, bi_v,
               utb, itb, ubb, ibb, outb, b16_v, sem0, sem1):
    wid = lax.axis_index("s") * NC + lax.axis_index("c")
    base = wid * BPW
    _stage_ids(uid2_h, iid2_h, uidq_v, iidq_v, uid_v, iid_v, wid)
    pltpu.sync_copy(b16_h, b16_v)
    for k in range(BPW // L):
        bu_v[pl.ds(k * L, L)] = lax.shift_right_logical(
            uid_v[pl.ds(k * L, L)], 7)
        bi_v[pl.ds(k * L, L)] = lax.shift_right_logical(
            iid_v[pl.ds(k * L, L)], 7)
    sems = (sem0, sem1)
    diags = _diags()
    iot = lax.iota(jnp.int32, L)

    def copies(c, s):
        return (
            pltpu.make_async_copy(ut_h.at[uid_v.at[pl.ds(c * C, C)]],
                                  utb.at[s], sems[s]),
            pltpu.make_async_copy(it_h.at[iid_v.at[pl.ds(c * C, C)]],
                                  itb.at[s], sems[s]),
            pltpu.make_async_copy(ubp_h.at[bu_v.at[pl.ds(c * C, C)]],
                                  ubb.at[s], sems[s]),
            pltpu.make_async_copy(ibp_h.at[bi_v.at[pl.ds(c * C, C)]],
                                  ibb.at[s], sems[s]),
        )

    def compute(c, s):
        ut = utb.at[s]
        it = itb.at[s]
        for g in range(C // L):
            rows = iot + g * L

            def body(jb, accs):
                accs = list(accs)
                cb = jnp.full((L,), jb * L, jnp.int32)
                for k in range(L):
                    col = cb + diags[k]
                    u = plsc.load_gather(ut, [rows, col])
                    v = plsc.load_gather(it, [rows, col])
                    accs[k & 3] = accs[k & 3] + u * v
                return tuple(accs)

            zero = jnp.zeros((L,), jnp.float32)
            accs = lax.fori_loop(0, BERT_DIM // L, body,
                                 (zero, zero, zero, zero))
            a0, a1, a2, a3 = accs
            ucol = jnp.bitwise_and(uid_v[pl.ds(c * C + g * L, L)], 127)
            icol = jnp.bitwise_and(iid_v[pl.ds(c * C + g * L, L)], 127)
            bia = (plsc.load_gather(ubb.at[s], [rows, ucol])
                   + plsc.load_gather(ibb.at[s], [rows, icol]))
            outb[pl.ds(c * C + g * L, L)] = ((a0 + a1) + (a2 + a3)
                                             + bia + b16_v[...])

    for d in copies(0, 0):
        d.start()

    def chunk_pair(i, carry):
        c = 2 * i
        for d in copies(c + 1, 1):
            d.start()
        for d in copies(c, 0):
            d.wait()
        compute(c, 0)

        @pl.when(i < NCH // 2 - 1)
        def _():
            for d in copies(c + 2, 0):
                d.start()

        for d in copies(c + 1, 1):
            d.wait()
        compute(c + 1, 1)
        return carry

    lax.fori_loop(0, NCH // 2, chunk_pair, 0)

    pltpu.sync_copy(outb, out_h.at[pl.ds(base, BPW)])


def _emb_body(uid2_h, iid2_h, ue_h, ie_h, out_h,
              uidq_v, iidq_v, uid_v, iid_v, eu_v, ei_v,
              ueb, ieb, outb, sem0, sem1):
    wid = lax.axis_index("s") * NC + lax.axis_index("c")
    base = wid * BPW
    _stage_ids(uid2_h, iid2_h, uidq_v, iidq_v, uid_v, iid_v, wid)
    diags = _diags()
    iot = lax.iota(jnp.int32, L)
    sems = (sem0, sem1)

    for k in range(BPW // L):
        eu_v[pl.ds(k * L, L)] = lax.shift_right_logical(
            uid_v[pl.ds(k * L, L)], 2)
        ei_v[pl.ds(k * L, L)] = lax.shift_right_logical(
            iid_v[pl.ds(k * L, L)], 2)

    def copies(c, s):
        return (
            pltpu.make_async_copy(ue_h.at[eu_v.at[pl.ds(c * CB, CB)]],
                                  ueb.at[s], sems[s]),
            pltpu.make_async_copy(ie_h.at[ei_v.at[pl.ds(c * CB, CB)]],
                                  ieb.at[s], sems[s]),
        )

    def compute(c, s):
        ue = ueb.at[s]
        ie = ieb.at[s]

        def group(g, carry):
            rows = iot + g * L
            cu = jnp.left_shift(
                jnp.bitwise_and(uid_v[pl.ds(c * CB + g * L, L)], 3), 5)
            ci = jnp.left_shift(
                jnp.bitwise_and(iid_v[pl.ds(c * CB + g * L, L)], 3), 5)
            acc0 = jnp.zeros((L,), jnp.float32)
            acc1 = jnp.zeros((L,), jnp.float32)
            for jb in range(EMB_DIM // L):
                for k in range(L):
                    colu = cu + jb * L + diags[k]
                    coli = ci + jb * L + diags[k]
                    u = plsc.load_gather(ue, [rows, colu])
                    v = plsc.load_gather(ie, [rows, coli])
                    if k & 1:
                        acc1 = acc1 + u * v
                    else:
                        acc0 = acc0 + u * v
            outb[pl.ds(c * CB + g * L, L)] = acc0 + acc1
            return carry

        lax.fori_loop(0, CB // L, group, 0)

    for d in copies(0, 0):
        d.start()
    for c in range(NCHB):
        s = c & 1
        if c + 1 < NCHB:
            for d in copies(c + 1, 1 - s):
                d.start()
        for d in copies(c, s):
            d.wait()
        compute(c, s)

    pltpu.sync_copy(outb, out_h.at[pl.ds(base, BPW)])


def kernel(user_ids, item_ids, user_emb_w, item_emb_w, user_text_w,
           item_text_w, user_bias, item_bias, bias):
    ue128 = user_emb_w.reshape(-1, 4 * EMB_DIM)
    ie128 = item_emb_w.reshape(-1, 4 * EMB_DIM)
    ubp = jnp.pad(user_bias, (0, NB_PAD - N_ROWS)).reshape(-1, 128)
    ibp = jnp.pad(item_bias, (0, NB_PAD - N_ROWS)).reshape(-1, 128)
    uid2 = user_ids.reshape(128, 128)
    iid2 = item_ids.reshape(128, 128)
    bias16 = jnp.broadcast_to(bias, (L,))

    mesh = plsc.VectorSubcoreMesh(core_axis_name="c", subcore_axis_name="s",
                                  num_cores=NC, num_subcores=NS)
    out_t = jax.ShapeDtypeStruct((B,), jnp.float32)
    params = pltpu.CompilerParams(use_tc_tiling_on_sc=True,
                                  needs_layout_passes=False)

    text_run = pl.kernel(
        _text_body,
        out_type=out_t,
        mesh=mesh,
        compiler_params=params,
        scratch_types=[
            pltpu.VMEM((QR, 128), jnp.int32),
            pltpu.VMEM((QR, 128), jnp.int32),
            pltpu.VMEM((BPW,), jnp.int32),
            pltpu.VMEM((BPW,), jnp.int32),
            pltpu.VMEM((BPW,), jnp.int32),
            pltpu.VMEM((BPW,), jnp.int32),
            pltpu.VMEM((2, C, BERT_DIM), jnp.float32),
            pltpu.VMEM((2, C, BERT_DIM), jnp.float32),
            pltpu.VMEM((2, C, 128), jnp.float32),
            pltpu.VMEM((2, C, 128), jnp.float32),
            pltpu.VMEM((BPW,), jnp.float32),
            pltpu.VMEM((L,), jnp.float32),
            pltpu.SemaphoreType.DMA,
            pltpu.SemaphoreType.DMA,
        ],
    )
    emb_run = pl.kernel(
        _emb_body,
        out_type=out_t,
        mesh=mesh,
        compiler_params=params,
        scratch_types=[
            pltpu.VMEM((QR, 128), jnp.int32),
            pltpu.VMEM((QR, 128), jnp.int32),
            pltpu.VMEM((BPW,), jnp.int32),
            pltpu.VMEM((BPW,), jnp.int32),
            pltpu.VMEM((BPW,), jnp.int32),
            pltpu.VMEM((BPW,), jnp.int32),
            pltpu.VMEM((2, CB, 4 * EMB_DIM), jnp.float32),
            pltpu.VMEM((2, CB, 4 * EMB_DIM), jnp.float32),
            pltpu.VMEM((BPW,), jnp.float32),
            pltpu.SemaphoreType.DMA,
            pltpu.SemaphoreType.DMA,
        ],
    )
    out_text = text_run(uid2, iid2, user_text_w, item_text_w,
                        ubp, ibp, bias16)
    out_emb = emb_run(uid2, iid2, ue128, ie128)
    return (out_text + out_emb)[:, None]


# speedup vs baseline: 1.1400x; 1.1400x over previous
import jax
import jax.numpy as jnp
from jax import lax
from jax.experimental import pallas as pl
from jax.experimental.pallas import tpu as pltpu
from jax.experimental.pallas import tpu_sc as plsc

B = 16384
EMB_DIM = 32
BERT_DIM = 768
NC = 2
NS = 16
L = 16
NW = NC * NS
BPW = B // NW
C = 32
NCH = BPW // C


def _diags():
    iot = lax.iota(jnp.int32, L)
    return [jnp.where(iot + k >= L, iot + k - L, iot + k) for k in range(L)]


def _text_body(uid_h, iid_h, ut_h, it_h, ub_h, ib_h, b16_h,
               out_h, uo_h, io_h,
               uid_v, iid_v, utb, itb, ubb, ibb, outb, b16_v, sem0, sem1):
    wid = lax.axis_index("s") * NC + lax.axis_index("c")
    base = wid * BPW
    pltpu.sync_copy(uid_h.at[pl.ds(base, BPW)], uid_v)
    pltpu.sync_copy(iid_h.at[pl.ds(base, BPW)], iid_v)
    pltpu.sync_copy(b16_h, b16_v)
    sems = (sem0, sem1)
    diags = _diags()
    iot = lax.iota(jnp.int32, L)

    def copies(c, s):
        return (
            pltpu.make_async_copy(ut_h.at[uid_v.at[pl.ds(c * C, C)]],
                                  utb.at[s], sems[s]),
            pltpu.make_async_copy(it_h.at[iid_v.at[pl.ds(c * C, C)]],
                                  itb.at[s], sems[s]),
            pltpu.make_async_copy(ub_h.at[uid_v.at[pl.ds(c * C, C)]],
                                  ubb.at[s], sems[s]),
            pltpu.make_async_copy(ib_h.at[iid_v.at[pl.ds(c * C, C)]],
                                  ibb.at[s], sems[s]),
        )

    def compute(c, s):
        ut = utb.at[s]
        it = itb.at[s]
        for g in range(C // L):
            rows = iot + g * L

            def body(jb, accs):
                accs = list(accs)
                cb = jnp.full((L,), jb * L, jnp.int32)
                for k in range(L):
                    col = cb + diags[k]
                    u = plsc.load_gather(ut, [rows, col])
                    v = plsc.load_gather(it, [rows, col])
                    accs[k & 3] = accs[k & 3] + u * v
                return tuple(accs)

            zero = jnp.zeros((L,), jnp.float32)
            accs = lax.fori_loop(0, BERT_DIM // L, body,
                                 (zero, zero, zero, zero))
            a0, a1, a2, a3 = accs
            bia = ubb.at[s][pl.ds(g * L, L)] + ibb.at[s][pl.ds(g * L, L)]
            outb[pl.ds(c * C + g * L, L)] = ((a0 + a1) + (a2 + a3)
                                             + bia + b16_v[...])

    for d in copies(0, 0):
        d.start()

    def chunk_pair(i, carry):
        c = 2 * i
        for d in copies(c + 1, 1):
            d.start()
        for d in copies(c, 0):
            d.wait()
        compute(c, 0)

        @pl.when(i < NCH // 2 - 1)
        def _():
            for d in copies(c + 2, 0):
                d.start()

        for d in copies(c + 1, 1):
            d.wait()
        compute(c + 1, 1)
        return carry

    lax.fori_loop(0, NCH // 2, chunk_pair, 0)

    pltpu.sync_copy(outb, out_h.at[pl.ds(base, BPW)])
    pltpu.sync_copy(uid_v, uo_h.at[pl.ds(base, BPW)])
    pltpu.sync_copy(iid_v, io_h.at[pl.ds(base, BPW)])


def _emb_body(uid_h, iid_h, ue_h, ie_h, out_h,
              uid_v, iid_v, ueb, ieb, outb, sem0):
    wid = lax.axis_index("s") * NC + lax.axis_index("c")
    base = wid * BPW
    pltpu.sync_copy(uid_h.at[pl.ds(base, BPW)], uid_v)
    pltpu.sync_copy(iid_h.at[pl.ds(base, BPW)], iid_v)
    diags = _diags()
    iot = lax.iota(jnp.int32, L)

    ds = (
        pltpu.make_async_copy(ue_h.at[uid_v], ueb, sem0),
        pltpu.make_async_copy(ie_h.at[iid_v], ieb, sem0),
    )
    for d in ds:
        d.start()
    for d in ds:
        d.wait()

    def group(g, carry):
        rows = iot + g * L
        acc = jnp.zeros((L,), jnp.float32)
        for jb in range(EMB_DIM // L):
            cb = jnp.full((L,), jb * L, jnp.int32)
            for k in range(L):
                col = cb + diags[k]
                u = plsc.load_gather(ueb, [rows, col])
                v = plsc.load_gather(ieb, [rows, col])
                acc = acc + u * v
        outb[pl.ds(g * L, L)] = acc
        return carry

    lax.fori_loop(0, BPW // L, group, 0)

    pltpu.sync_copy(outb, out_h.at[pl.ds(base, BPW)])


def kernel(user_ids, item_ids, user_emb_w, item_emb_w, user_text_w,
           item_text_w, user_bias, item_bias, bias):
    mesh = plsc.VectorSubcoreMesh(core_axis_name="c", subcore_axis_name="s",
                                  num_cores=NC, num_subcores=NS)
    out_t = jax.ShapeDtypeStruct((B,), jnp.float32)
    id_t = jax.ShapeDtypeStruct((B,), jnp.int32)

    text_run = pl.kernel(
        _text_body,
        out_type=(out_t, id_t, id_t),
        mesh=mesh,
        compiler_params=pltpu.CompilerParams(use_tc_tiling_on_sc=True,
                                             needs_layout_passes=False),
        scratch_types=[
            pltpu.VMEM((BPW,), jnp.int32),
            pltpu.VMEM((BPW,), jnp.int32),
            pltpu.VMEM((2, C, BERT_DIM), jnp.float32),
            pltpu.VMEM((2, C, BERT_DIM), jnp.float32),
            pltpu.VMEM((2, C), jnp.float32),
            pltpu.VMEM((2, C), jnp.float32),
            pltpu.VMEM((BPW,), jnp.float32),
            pltpu.VMEM((L,), jnp.float32),
            pltpu.SemaphoreType.DMA,
            pltpu.SemaphoreType.DMA,
        ],
    )
    emb_run = pl.kernel(
        _emb_body,
        out_type=out_t,
        mesh=mesh,
        compiler_params=pltpu.CompilerParams(use_tc_tiling_on_sc=False,
                                             needs_layout_passes=False),
        scratch_types=[
            pltpu.VMEM((BPW,), jnp.int32),
            pltpu.VMEM((BPW,), jnp.int32),
            pltpu.VMEM((BPW, EMB_DIM), jnp.float32),
            pltpu.VMEM((BPW, EMB_DIM), jnp.float32),
            pltpu.VMEM((BPW,), jnp.float32),
            pltpu.SemaphoreType.DMA,
        ],
    )
    bias16 = jnp.broadcast_to(bias, (L,))
    out_text, uo, io = text_run(user_ids, item_ids, user_text_w,
                                item_text_w, user_bias, item_bias, bias16)
    out_emb = emb_run(uo, io, user_emb_w, item_emb_w)
    return (out_text + out_emb)[:, None]
